# baseline (speedup 1.0000x reference)
import functools

import jax
import jax.numpy as jnp
from jax import lax
from jax.experimental import pallas as pl
from jax.experimental.pallas import tpu as pltpu
from jax.experimental.pallas import tpu_sc as plsc

_D = 64
_C = 400


@functools.cache
def _lookup_fn(n):
    info = plsc.get_sparse_core_info()
    nw = info.num_cores * info.num_subcores
    per_w = n // nw
    chunks = per_w // _C
    pairs = chunks // 2
    assert per_w * nw == n and pairs * 2 * _C == per_w

    mesh = plsc.VectorSubcoreMesh(core_axis_name="c", subcore_axis_name="s")

    @functools.partial(
        pl.kernel,
        mesh=mesh,
        compiler_params=pltpu.CompilerParams(use_tc_tiling_on_sc=False),
        out_type=jax.ShapeDtypeStruct((n, 2 * _D), jnp.float32),
        scratch_types=[
            [pltpu.VMEM((_C,), jnp.int32) for _ in range(2)],
            [pltpu.VMEM((_C,), jnp.int32) for _ in range(2)],
            [pltpu.VMEM((_C, _D), jnp.float32) for _ in range(2)],
            [pltpu.VMEM((_C, _D), jnp.float32) for _ in range(2)],
            [pltpu.SemaphoreType.DMA for _ in range(2)],
            [pltpu.SemaphoreType.DMA for _ in range(2)],
            [pltpu.SemaphoreType.DMA for _ in range(2)],
            [pltpu.SemaphoreType.DMA for _ in range(2)],
        ],
    )
    def k(tok, pos, ttab, ptab, out,
          idx_t, idx_p, rows_t, rows_p, gsem, wsem_t, wsem_p, isem):
        wid = lax.axis_index("s") * info.num_cores + lax.axis_index("c")
        w_base = wid * per_w
        last_base = w_base + per_w - _C

        def fire_idx(i, s):
            base = jnp.minimum(w_base + i * _C, last_base)
            pltpu.async_copy(tok.at[pl.ds(base, _C)], idx_t[s], isem[s])
            pltpu.async_copy(pos.at[pl.ds(base, _C)], idx_p[s], isem[s])

        def drain_idx(s):
            pltpu.make_async_copy(tok.at[pl.ds(0, _C)], idx_t[s], isem[s]).wait()
            pltpu.make_async_copy(pos.at[pl.ds(0, _C)], idx_p[s], isem[s]).wait()

        def fire_writes(i, s):
            base = w_base + i * _C
            pltpu.async_copy(rows_t[s], out.at[pl.ds(base, _C), pl.ds(0, _D)], wsem_t[s])
            pltpu.async_copy(rows_p[s], out.at[pl.ds(base, _C), pl.ds(_D, _D)], wsem_p[s])

        def drain_writes(s):
            pltpu.make_async_copy(
                rows_t[s], out.at[pl.ds(0, _C), pl.ds(0, _D)], wsem_t[s]).wait()
            pltpu.make_async_copy(
                rows_p[s], out.at[pl.ds(0, _C), pl.ds(_D, _D)], wsem_p[s]).wait()

        for s in range(2):
            fire_idx(s, s)
        for s in range(2):
            drain_idx(s)
            ct = pltpu.async_copy(ttab.at[idx_t[s]], rows_t[s], gsem[s])
            cp = pltpu.async_copy(ptab.at[idx_p[s]], rows_p[s], gsem[s])
            ct.wait()
            cp.wait()
            fire_idx(2 + s, s)
            fire_writes(s, s)

        def body(j, carry):
            copies = []
            for s in range(2):
                drain_idx(s)
                drain_writes(s)
                ct = pltpu.async_copy(ttab.at[idx_t[s]], rows_t[s], gsem[s])
                cp = pltpu.async_copy(ptab.at[idx_p[s]], rows_p[s], gsem[s])
                copies.append((ct, cp))
            for s in range(2):
                ct, cp = copies[s]
                ct.wait()
                cp.wait()
                fire_idx(2 * j + s + 2, s)
                fire_writes(2 * j + s, s)
            return carry

        lax.fori_loop(1, pairs, body, 0)
        for s in range(2):
            drain_idx(s)
            drain_writes(s)

    return k


def kernel(tokens, pos, token_table, pos_table):
    B, L = tokens.shape
    n = B * L
    fn = _lookup_fn(n)
    out = fn(tokens.reshape(n), pos.reshape(n), token_table, pos_table)
    return out.reshape(B, L, 2 * _D)


# speedup vs baseline: 1.0006x; 1.0006x over previous
import functools

import jax
import jax.numpy as jnp
from jax import lax
from jax.experimental import pallas as pl
from jax.experimental.pallas import tpu as pltpu
from jax.experimental.pallas import tpu_sc as plsc

_D = 64
_C = 400


@functools.cache
def _lookup_fn(n):
    info = plsc.get_sparse_core_info()
    nw = info.num_cores * info.num_subcores
    per_w = n // nw
    chunks = per_w // _C
    pairs = chunks // 2
    assert per_w * nw == n and pairs * 2 * _C == per_w

    mesh = plsc.VectorSubcoreMesh(core_axis_name="c", subcore_axis_name="s")

    @functools.partial(
        pl.kernel,
        mesh=mesh,
        compiler_params=pltpu.CompilerParams(use_tc_tiling_on_sc=False),
        out_type=jax.ShapeDtypeStruct((n, 2 * _D), jnp.float32),
        scratch_types=[
            [pltpu.VMEM((_C,), jnp.int32) for _ in range(2)],
            [pltpu.VMEM((_C,), jnp.int32) for _ in range(2)],
            [pltpu.VMEM((_C, _D), jnp.float32) for _ in range(2)],
            [pltpu.VMEM((_C, _D), jnp.float32) for _ in range(2)],
            [pltpu.SemaphoreType.DMA for _ in range(2)],
            [pltpu.SemaphoreType.DMA for _ in range(2)],
            [pltpu.SemaphoreType.DMA for _ in range(2)],
            [pltpu.SemaphoreType.DMA for _ in range(2)],
        ],
    )
    def k(tok, pos, ttab, ptab, out,
          idx_t, idx_p, rows_t, rows_p, gsem, wsem_t, wsem_p, isem):
        wid = lax.axis_index("s") * info.num_cores + lax.axis_index("c")
        w_base = wid * per_w
        last_base = w_base + per_w - _C

        def fire_idx(i, s):
            base = jnp.minimum(w_base + i * _C, last_base)
            pltpu.async_copy(tok.at[pl.ds(base, _C)], idx_t[s], isem[s])
            pltpu.async_copy(pos.at[pl.ds(base, _C)], idx_p[s], isem[s])

        def drain_idx(s):
            pltpu.make_async_copy(tok.at[pl.ds(0, _C)], idx_t[s], isem[s]).wait()
            pltpu.make_async_copy(pos.at[pl.ds(0, _C)], idx_p[s], isem[s]).wait()

        def fire_writes(i, s):
            base = w_base + i * _C
            pltpu.async_copy(rows_t[s], out.at[pl.ds(base, _C), pl.ds(0, _D)], wsem_t[s])
            pltpu.async_copy(rows_p[s], out.at[pl.ds(base, _C), pl.ds(_D, _D)], wsem_p[s])

        def drain_writes(s):
            pltpu.make_async_copy(
                rows_t[s], out.at[pl.ds(0, _C), pl.ds(0, _D)], wsem_t[s]).wait()
            pltpu.make_async_copy(
                rows_p[s], out.at[pl.ds(0, _C), pl.ds(_D, _D)], wsem_p[s]).wait()

        for s in range(2):
            fire_idx(s, s)
        for s in range(2):
            drain_idx(s)
            ct = pltpu.async_copy(ttab.at[idx_t[s]], rows_t[s], gsem[s])
            cp = pltpu.async_copy(ptab.at[idx_p[s]], rows_p[s], gsem[s])
            ct.wait()
            cp.wait()
            fire_idx(2 + s, s)
            fire_writes(s, s)

        def body(j, carry):
            for s in range(2):
                i = 2 * j + s
                drain_idx(s)
                drain_writes(s)
                ct = pltpu.async_copy(ttab.at[idx_t[s]], rows_t[s], gsem[s])
                cp = pltpu.async_copy(ptab.at[idx_p[s]], rows_p[s], gsem[s])
                ct.wait()
                cp.wait()
                fire_idx(i + 2, s)
                fire_writes(i, s)
            return carry

        lax.fori_loop(1, pairs, body, 0)
        for s in range(2):
            drain_idx(s)
            drain_writes(s)

    return k


def kernel(tokens, pos, token_table, pos_table):
    B, L = tokens.shape
    n = B * L
    fn = _lookup_fn(n)
    out = fn(tokens.reshape(n), pos.reshape(n), token_table, pos_table)
    return out.reshape(B, L, 2 * _D)
